# baseline (speedup 1.0000x reference)
import functools

import jax
import jax.numpy as jnp
from jax import lax
from jax.experimental import pallas as pl
from jax.experimental.pallas import tpu as pltpu
from jax.experimental.pallas import tpu_sc as plsc

NB_USER_ROWS = 1000000
NB_ITEM_ROWS = 1000000
F = 64
B = 16384

NC = 2
NS = 16
L = 16
NW = NC * NS
B_PER_W = B // NW
CHUNK = 128
N_CHUNKS = B_PER_W // CHUNK
GROUPS = CHUNK // L

_MESH = plsc.VectorSubcoreMesh(
    core_axis_name="c", subcore_axis_name="s", num_cores=NC, num_subcores=NS
)


@functools.partial(
    pl.kernel,
    out_type=jax.ShapeDtypeStruct((B,), jnp.float32),
    mesh=_MESH,
    compiler_params=pltpu.CompilerParams(
        needs_layout_passes=False, use_tc_tiling_on_sc=False
    ),
    scratch_types=[
        pltpu.VMEM((B_PER_W,), jnp.int32),
        pltpu.VMEM((B_PER_W,), jnp.int32),
        pltpu.VMEM((CHUNK, F), jnp.float32),
        pltpu.VMEM((CHUNK, F), jnp.float32),
        pltpu.VMEM((L,), jnp.float32),
        pltpu.VMEM((B_PER_W,), jnp.float32),
        pltpu.SemaphoreType.DMA,
    ],
)
def _mf_sc(users_hbm, items_hbm, ut_hbm, it_hbm, gb_hbm,
           out_hbm, uidx, iidx, urows, irows, gbv, outv, sem):
    wid = lax.axis_index("s") * NC + lax.axis_index("c")
    base = wid * B_PER_W

    pltpu.sync_copy(users_hbm.at[pl.ds(base, B_PER_W)], uidx)
    pltpu.sync_copy(items_hbm.at[pl.ds(base, B_PER_W)], iidx)
    pltpu.sync_copy(gb_hbm, gbv)
    gvec = gbv[...]

    for c in range(N_CHUNKS):
        uslice = uidx.at[pl.ds(c * CHUNK, CHUNK)]
        islice = iidx.at[pl.ds(c * CHUNK, CHUNK)]
        d1 = pltpu.async_copy(ut_hbm.at[uslice], urows, sem)
        d2 = pltpu.async_copy(it_hbm.at[islice], irows, sem)
        d1.wait()
        d2.wait()

        def group_body(g, _, c=c):
            rows = lax.iota(jnp.int32, L) + g * L
            cols = jnp.zeros((L,), jnp.int32)
            acc = gvec
            for _f in range(F):
                uv = plsc.load_gather(urows, [rows, cols])
                iv = plsc.load_gather(irows, [rows, cols])
                acc = acc + uv * iv
                cols = cols + 1
            outv[pl.ds(c * CHUNK + g * L, L)] = acc
            return 0

        lax.fori_loop(0, GROUPS, group_body, 0)

    pltpu.sync_copy(outv, out_hbm.at[pl.ds(base, B_PER_W)])


def kernel(users, items, user_table, item_table, user_bias, item_bias, global_bias):
    gb = jnp.broadcast_to(global_bias.reshape((1,)), (L,))
    out = _mf_sc(users, items, user_table, item_table, gb)
    out = out.reshape((B, 1))
    return (out + jnp.take(user_bias, users, axis=0)
            + jnp.take(item_bias, items, axis=0))


# speedup vs baseline: 1.3851x; 1.3851x over previous
import functools

import jax
import jax.numpy as jnp
from jax import lax
from jax.experimental import pallas as pl
from jax.experimental.pallas import tpu as pltpu
from jax.experimental.pallas import tpu_sc as plsc

NB_ROWS = 1000000
F = 64
B = 16384

NC = 2
NS = 16
L = 16
NW = NC * NS
B_PER_W = B // NW
GROUPS = B_PER_W // L
PAIRS = GROUPS // 2

_MESH = plsc.VectorSubcoreMesh(
    core_axis_name="c", subcore_axis_name="s", num_cores=NC, num_subcores=NS
)


@functools.partial(
    pl.kernel,
    out_type=jax.ShapeDtypeStruct((B,), jnp.float32),
    mesh=_MESH,
    compiler_params=pltpu.CompilerParams(
        needs_layout_passes=False, use_tc_tiling_on_sc=True
    ),
    scratch_types=[
        pltpu.VMEM((B_PER_W,), jnp.int32),
        pltpu.VMEM((B_PER_W,), jnp.int32),
        pltpu.VMEM((4 * L, F), jnp.float32),
        pltpu.VMEM((4 * L, F), jnp.float32),
        pltpu.VMEM((L,), jnp.float32),
        pltpu.VMEM((B_PER_W,), jnp.float32),
        pltpu.SemaphoreType.DMA,
        pltpu.SemaphoreType.DMA,
        pltpu.SemaphoreType.DMA,
        pltpu.SemaphoreType.DMA,
    ],
)
def _mf_sc(users_hbm, items_hbm, ut_hbm, it_hbm, gb_hbm,
           out_hbm, uidx, iidx, urows, irows, gbv, outv,
           sem0, sem1, sem2, sem3):
    wid = lax.axis_index("s") * NC + lax.axis_index("c")
    base = wid * B_PER_W

    pltpu.sync_copy(users_hbm.at[pl.ds(base, B_PER_W)], uidx)
    pltpu.sync_copy(items_hbm.at[pl.ds(base, B_PER_W)], iidx)
    pltpu.sync_copy(gb_hbm, gbv)
    gvec = gbv[...]

    def issue(g, sem):
        vu = uidx[pl.ds(g * L, L)]
        vi = iidx[pl.ds(g * L, L)]
        band = lax.bitwise_and(g, 3) * L
        for j in range(L):
            ru = vu[j]
            ri = vi[j]
            slot = band + j
            pltpu.async_copy(ut_hbm.at[ru], urows.at[slot], sem)
            pltpu.async_copy(it_hbm.at[ri], irows.at[slot], sem)

    def drain(g, sem):
        band = lax.bitwise_and(g, 3) * L
        pltpu.make_async_copy(
            ut_hbm.at[pl.ds(0, L)], urows.at[pl.ds(band, L)], sem).wait()
        pltpu.make_async_copy(
            it_hbm.at[pl.ds(0, L)], irows.at[pl.ds(band, L)], sem).wait()

    def compute(g):
        rows = lax.iota(jnp.int32, L) + lax.bitwise_and(g, 3) * L
        cols = jnp.zeros((L,), jnp.int32)
        acc = gvec
        for _f in range(F):
            uv = plsc.load_gather(urows, [rows, cols])
            iv = plsc.load_gather(irows, [rows, cols])
            acc = acc + uv * iv
            cols = cols + 1
        outv[pl.ds(g * L, L)] = acc

    def body(p, _):
        sa, sb = (sem0, sem1), (sem2, sem3)
        parity = lax.rem(p, 2)

        @pl.when(parity == 0)
        def _():
            issue(2 * p, sa[0])
            issue(2 * p + 1, sa[1])

        @pl.when(parity == 1)
        def _():
            issue(2 * p, sb[0])
            issue(2 * p + 1, sb[1])

        @pl.when(jnp.logical_and(p > 0, parity == 1))
        def _():
            drain(2 * p - 2, sa[0])
            compute(2 * p - 2)
            drain(2 * p - 1, sa[1])
            compute(2 * p - 1)

        @pl.when(jnp.logical_and(p > 0, parity == 0))
        def _():
            drain(2 * p - 2, sb[0])
            compute(2 * p - 2)
            drain(2 * p - 1, sb[1])
            compute(2 * p - 1)

        return 0

    lax.fori_loop(0, PAIRS, body, 0)
    last = PAIRS - 1
    fs = (sem0, sem1) if last % 2 == 0 else (sem2, sem3)
    drain(2 * last, fs[0])
    compute(2 * last)
    drain(2 * last + 1, fs[1])
    compute(2 * last + 1)

    pltpu.sync_copy(outv, out_hbm.at[pl.ds(base, B_PER_W)])


def kernel(users, items, user_table, item_table, user_bias, item_bias, global_bias):
    gb = jnp.broadcast_to(global_bias.reshape((1,)), (L,))
    out = _mf_sc(users, items, user_table, item_table, gb)
    out = out.reshape((B, 1))
    return (out + jnp.take(user_bias, users, axis=0)
            + jnp.take(item_bias, items, axis=0))
